# Initial kernel scaffold; baseline (speedup 1.0000x reference)
import functools

import jax
import jax.numpy as jnp
from jax import lax
from jax.experimental import pallas as pl
from jax.experimental.pallas import tpu as pltpu
from jax.experimental.pallas import tpu_sc as plsc

N = 10000
E = 320000
D = 128

NC = 2
NS = 16
NW = NC * NS
EPW = E // NW
K = 80
NCHUNK = EPW // K
RPT = N // NS
ZR = 125
CW = 16


def _agg_body(with_counts, *refs):
    if with_counts:
        (y_hbm, src_hbm, dst_hbm, out_hbm, cnt_hbm,
         sidx, didx, rows, zbuf, acc, sem, ones, czbuf, cacc) = refs
    else:
        (y_hbm, src_hbm, dst_hbm, out_hbm,
         sidx, didx, rows, zbuf, acc, sem) = refs

    core = lax.axis_index("c")
    sub = lax.axis_index("s")
    wid = core * NS + sub

    zero16 = jnp.zeros((16,), jnp.float32)

    def zrow(i, c):
        for j in range(D // 16):
            zbuf[i, pl.ds(j * 16, 16)] = zero16
        return c
    lax.fori_loop(0, ZR, zrow, 0)

    r0 = sub * RPT
    for t in range(RPT // ZR):
        pltpu.sync_copy(zbuf, acc.at[pl.ds(r0 + t * ZR, ZR)])

    if with_counts:
        one16 = jnp.ones((16,), jnp.float32)

        def crow(i, c):
            czbuf[i, :] = zero16
            return c
        lax.fori_loop(0, RPT, crow, 0)
        pltpu.sync_copy(czbuf, cacc.at[pl.ds(r0, RPT)])

        def orow(i, c):
            ones[i, :] = one16
            return c
        lax.fori_loop(0, K, orow, 0)

    plsc.subcore_barrier()

    g0 = wid * EPW

    def chunk(j, c):
        base = g0 + j * K
        pltpu.sync_copy(src_hbm.at[pl.ds(base, K)], sidx)
        pltpu.sync_copy(dst_hbm.at[pl.ds(base, K)], didx)
        pltpu.async_copy(y_hbm.at[sidx], rows, sem).wait()
        pltpu.sync_copy(rows, acc.at[didx], add=True)
        if with_counts:
            pltpu.sync_copy(ones, cacc.at[didx], add=True)
        return c
    lax.fori_loop(0, NCHUNK, chunk, 0)

    plsc.subcore_barrier()

    pltpu.sync_copy(acc.at[pl.ds(r0, RPT)], out_hbm.at[core, pl.ds(r0, RPT)])
    if with_counts:
        pltpu.sync_copy(cacc.at[pl.ds(r0, RPT)], cnt_hbm.at[core, pl.ds(r0, RPT)])


def _make_agg(with_counts):
    mesh = plsc.VectorSubcoreMesh(core_axis_name="c", subcore_axis_name="s")
    out_type = [jax.ShapeDtypeStruct((NC, N, D), jnp.float32)]
    scratch = [
        pltpu.VMEM((K,), jnp.int32),
        pltpu.VMEM((K,), jnp.int32),
        pltpu.VMEM((K, D), jnp.float32),
        pltpu.VMEM((ZR, D), jnp.float32),
        pltpu.VMEM_SHARED((N, D), jnp.float32),
        pltpu.SemaphoreType.DMA,
    ]
    if with_counts:
        out_type.append(jax.ShapeDtypeStruct((NC, N, CW), jnp.float32))
        scratch += [
            pltpu.VMEM((K, CW), jnp.float32),
            pltpu.VMEM((RPT, CW), jnp.float32),
            pltpu.VMEM_SHARED((N, CW), jnp.float32),
        ]
    return pl.kernel(
        functools.partial(_agg_body, with_counts),
        out_type=out_type,
        mesh=mesh,
        scratch_types=scratch,
    )


_agg_with_counts = _make_agg(True)
_agg_no_counts = _make_agg(False)


_RB = 500
_NG = N // _RB

_full_w = pl.BlockSpec((D, D), lambda i: (0, 0))
_full_b = pl.BlockSpec((1, D), lambda i: (0, 0))
_row_blk = pl.BlockSpec((_RB, D), lambda i: (i, 0))
_agg_blk = pl.BlockSpec((NC, _RB, D), lambda i: (0, i, 0))
_cnt_blk = pl.BlockSpec((NC, _RB, CW), lambda i: (0, i, 0))


def _pre_body(x_ref, wl_ref, wr_ref, b_ref, y_ref, s_ref):
    xb = x_ref[...]
    y_ref[...] = jnp.dot(xb, wl_ref[...], preferred_element_type=jnp.float32)
    s_ref[...] = (jnp.dot(xb, wr_ref[...], preferred_element_type=jnp.float32)
                  + b_ref[...])


def _pre(x, wl, wr, b):
    return pl.pallas_call(
        _pre_body,
        grid=(_NG,),
        in_specs=[_row_blk, _full_w, _full_w, _full_b],
        out_specs=[_row_blk, _row_blk],
        out_shape=[jax.ShapeDtypeStruct((N, D), jnp.float32),
                   jax.ShapeDtypeStruct((N, D), jnp.float32)],
    )(x, wl, wr, b)


def _mid_body(agg_ref, cnt_ref, s_ref, wl_ref, wr_ref, b_ref, y_ref, s2_ref):
    a = agg_ref[0] + agg_ref[1]
    cn = cnt_ref[0, :, 0:1] + cnt_ref[1, :, 0:1]
    rinv = 1.0 / jnp.maximum(cn, 1.0)
    z = jnp.maximum(a * rinv + s_ref[...], 0.0)
    y_ref[...] = jnp.dot(z, wl_ref[...], preferred_element_type=jnp.float32)
    s2_ref[...] = (jnp.dot(z, wr_ref[...], preferred_element_type=jnp.float32)
                   + b_ref[...])


def _mid(agg, cnt, s1, wl, wr, b):
    return pl.pallas_call(
        _mid_body,
        grid=(_NG,),
        in_specs=[_agg_blk, _cnt_blk, _row_blk, _full_w, _full_w, _full_b],
        out_specs=[_row_blk, _row_blk],
        out_shape=[jax.ShapeDtypeStruct((N, D), jnp.float32),
                   jax.ShapeDtypeStruct((N, D), jnp.float32)],
    )(agg, cnt, s1, wl, wr, b)


def _fin_body(agg_ref, cnt_ref, s_ref, o_ref):
    a = agg_ref[0] + agg_ref[1]
    cn = cnt_ref[0, :, 0:1] + cnt_ref[1, :, 0:1]
    rinv = 1.0 / jnp.maximum(cn, 1.0)
    o_ref[...] = a * rinv + s_ref[...]


def _fin(agg, cnt, s2):
    return pl.pallas_call(
        _fin_body,
        grid=(_NG,),
        in_specs=[_agg_blk, _cnt_blk, _row_blk],
        out_specs=_row_blk,
        out_shape=jax.ShapeDtypeStruct((N, D), jnp.float32),
    )(agg, cnt, s2)


@jax.jit
def kernel(x, edge_index, W_l1, b_l1, W_r1, W_l2, b_l2, W_r2):
    src = edge_index[0]
    dst = edge_index[1]
    y1, s1 = _pre(x, W_l1, W_r1, b_l1.reshape(1, D))
    agg1, cnt = _agg_with_counts(y1, src, dst)
    y2, s2 = _mid(agg1, cnt, s1, W_l2, W_r2, b_l2.reshape(1, D))
    (agg2,) = _agg_no_counts(y2, src, dst)
    return _fin(agg2, cnt, s2)


# speedup vs baseline: 5.4354x; 5.4354x over previous
import functools

import jax
import jax.numpy as jnp
from jax import lax
from jax.experimental import pallas as pl
from jax.experimental.pallas import tpu as pltpu
from jax.experimental.pallas import tpu_sc as plsc

N = 10000
E = 320000
D = 128

NC = 2
NS = 16
NW = NC * NS
EPW = E // NW
K = 80
NCHUNK = EPW // K
NP = 10240
RPT = NP // NS
ZR = 128
CW = 16


def _agg_body(with_counts, *refs):
    if with_counts:
        (y_hbm, src_hbm, dst_hbm, out_hbm, cnt_hbm,
         sidx, didx, rows, zbuf, acc, sem, ones, czbuf, cacc) = refs
    else:
        (y_hbm, src_hbm, dst_hbm, out_hbm,
         sidx, didx, rows, zbuf, acc, sem) = refs

    core = lax.axis_index("c")
    sub = lax.axis_index("s")
    wid = core * NS + sub

    zero16 = jnp.zeros((16,), jnp.float32)

    def zrow(i, c):
        for j in range(D // 16):
            zbuf[i, pl.ds(j * 16, 16)] = zero16
        return c
    lax.fori_loop(0, ZR, zrow, 0)

    r0 = sub * RPT
    for t in range(RPT // ZR):
        pltpu.sync_copy(zbuf, acc.at[pl.ds(r0 + t * ZR, ZR)])

    if with_counts:
        one16 = jnp.ones((16,), jnp.float32)

        def crow(i, c):
            czbuf[i, :] = zero16
            return c
        lax.fori_loop(0, RPT, crow, 0)
        pltpu.sync_copy(czbuf, cacc.at[pl.ds(r0, RPT)])

        def orow(i, c):
            ones[i, :] = one16
            return c
        lax.fori_loop(0, K, orow, 0)

    plsc.subcore_barrier()

    g0 = wid * EPW

    def chunk(j, c):
        base = g0 + j * K
        pltpu.sync_copy(src_hbm.at[pl.ds(base, K)], sidx)
        pltpu.sync_copy(dst_hbm.at[pl.ds(base, K)], didx)
        pltpu.async_copy(y_hbm.at[sidx], rows, sem).wait()
        pltpu.sync_copy(rows, acc.at[didx], add=True)
        if with_counts:
            pltpu.sync_copy(ones, cacc.at[didx], add=True)
        return c
    lax.fori_loop(0, NCHUNK, chunk, 0)

    plsc.subcore_barrier()

    pltpu.sync_copy(acc.at[pl.ds(r0, RPT)], out_hbm.at[core, pl.ds(r0, RPT)])
    if with_counts:
        pltpu.sync_copy(cacc.at[pl.ds(r0, RPT)], cnt_hbm.at[core, pl.ds(r0, RPT)])


def _make_agg(with_counts):
    mesh = plsc.VectorSubcoreMesh(core_axis_name="c", subcore_axis_name="s")
    out_type = [jax.ShapeDtypeStruct((NC, NP, D), jnp.float32)]
    scratch = [
        pltpu.VMEM((K,), jnp.int32),
        pltpu.VMEM((K,), jnp.int32),
        pltpu.VMEM((K, D), jnp.float32),
        pltpu.VMEM((ZR, D), jnp.float32),
        pltpu.VMEM_SHARED((NP, D), jnp.float32),
        pltpu.SemaphoreType.DMA,
    ]
    if with_counts:
        out_type.append(jax.ShapeDtypeStruct((NC, NP, CW), jnp.float32))
        scratch += [
            pltpu.VMEM((K, CW), jnp.float32),
            pltpu.VMEM((RPT, CW), jnp.float32),
            pltpu.VMEM_SHARED((NP, CW), jnp.float32),
        ]
    return pl.kernel(
        functools.partial(_agg_body, with_counts),
        out_type=out_type,
        mesh=mesh,
        scratch_types=scratch,
        compiler_params=pltpu.CompilerParams(use_tc_tiling_on_sc=False),
    )


_agg_with_counts = _make_agg(True)
_agg_no_counts = _make_agg(False)


_RB = 1000
_NG = N // _RB

_full_w = pl.BlockSpec((D, D), lambda i: (0, 0))
_full_b = pl.BlockSpec((1, D), lambda i: (0, 0))
_row_blk = pl.BlockSpec((_RB, D), lambda i: (i, 0))
_agg_blk = pl.BlockSpec((NC, _RB, D), lambda i: (0, i, 0))
_cnt_blk = pl.BlockSpec((NC, _RB, CW), lambda i: (0, i, 0))


def _pre_body(x_ref, wl_ref, wr_ref, b_ref, y_ref, s_ref):
    xb = x_ref[...]
    y_ref[...] = jnp.dot(xb, wl_ref[...], preferred_element_type=jnp.float32)
    s_ref[...] = (jnp.dot(xb, wr_ref[...], preferred_element_type=jnp.float32)
                  + b_ref[...])


def _pre(x, wl, wr, b):
    return pl.pallas_call(
        _pre_body,
        grid=(_NG,),
        in_specs=[_row_blk, _full_w, _full_w, _full_b],
        out_specs=[_row_blk, _row_blk],
        out_shape=[jax.ShapeDtypeStruct((N, D), jnp.float32),
                   jax.ShapeDtypeStruct((N, D), jnp.float32)],
    )(x, wl, wr, b)


def _mid_body(agg_ref, cnt_ref, s_ref, wl_ref, wr_ref, b_ref, y_ref, s2_ref):
    a = agg_ref[0] + agg_ref[1]
    cn = cnt_ref[0, :, 0:1] + cnt_ref[1, :, 0:1]
    rinv = 1.0 / jnp.maximum(cn, 1.0)
    z = jnp.maximum(a * rinv + s_ref[...], 0.0)
    y_ref[...] = jnp.dot(z, wl_ref[...], preferred_element_type=jnp.float32)
    s2_ref[...] = (jnp.dot(z, wr_ref[...], preferred_element_type=jnp.float32)
                   + b_ref[...])


def _mid(agg, cnt, s1, wl, wr, b):
    return pl.pallas_call(
        _mid_body,
        grid=(_NG,),
        in_specs=[_agg_blk, _cnt_blk, _row_blk, _full_w, _full_w, _full_b],
        out_specs=[_row_blk, _row_blk],
        out_shape=[jax.ShapeDtypeStruct((N, D), jnp.float32),
                   jax.ShapeDtypeStruct((N, D), jnp.float32)],
    )(agg, cnt, s1, wl, wr, b)


def _fin_body(agg_ref, cnt_ref, s_ref, o_ref):
    a = agg_ref[0] + agg_ref[1]
    cn = cnt_ref[0, :, 0:1] + cnt_ref[1, :, 0:1]
    rinv = 1.0 / jnp.maximum(cn, 1.0)
    o_ref[...] = a * rinv + s_ref[...]


def _fin(agg, cnt, s2):
    return pl.pallas_call(
        _fin_body,
        grid=(_NG,),
        in_specs=[_agg_blk, _cnt_blk, _row_blk],
        out_specs=_row_blk,
        out_shape=jax.ShapeDtypeStruct((N, D), jnp.float32),
    )(agg, cnt, s2)


@jax.jit
def kernel(x, edge_index, W_l1, b_l1, W_r1, W_l2, b_l2, W_r2):
    src = edge_index[0]
    dst = edge_index[1]
    y1, s1 = _pre(x, W_l1, W_r1, b_l1.reshape(1, D))
    agg1, cnt = _agg_with_counts(y1, src, dst)
    y2, s2 = _mid(agg1, cnt, s1, W_l2, W_r2, b_l2.reshape(1, D))
    (agg2,) = _agg_no_counts(y2, src, dst)
    return _fin(agg2, cnt, s2)


# speedup vs baseline: 8.6483x; 1.5911x over previous
import functools

import jax
import jax.numpy as jnp
from jax import lax
from jax.experimental import pallas as pl
from jax.experimental.pallas import tpu as pltpu
from jax.experimental.pallas import tpu_sc as plsc

N = 10000
E = 320000
D = 128

NC = 2
NS = 16
NW = NC * NS
EPW = E // NW
K = 80
NCHUNK = EPW // K
NP = 10240
RPT = NP // NS
ZR = 128
CW = 16


def _agg_body(with_counts, *refs):
    if with_counts:
        (y_hbm, src_hbm, dst_hbm, out_hbm, cnt_hbm,
         sidx, didx, rows, acc, sem, csem, ones, czbuf, cacc) = refs
    else:
        (y_hbm, src_hbm, dst_hbm, out_hbm,
         sidx, didx, rows, acc, sem) = refs

    core = lax.axis_index("c")
    sub = lax.axis_index("s")
    wid = core * NS + sub

    zero16 = jnp.zeros((16,), jnp.float32)

    for b in range(2):
        def zrow(i, c, _b=b):
            for j in range(D // 16):
                rows[_b, i, pl.ds(j * 16, 16)] = zero16
            return c
        lax.fori_loop(0, K, zrow, 0)

    r0 = sub * RPT
    for t in range(RPT // K):
        pltpu.sync_copy(rows.at[t % 2], acc.at[pl.ds(r0 + t * K, K)])

    if with_counts:
        one16 = jnp.ones((16,), jnp.float32)

        def crow(i, c):
            czbuf[i, :] = zero16
            return c
        lax.fori_loop(0, K, crow, 0)
        for t in range(RPT // K):
            pltpu.sync_copy(czbuf, cacc.at[pl.ds(r0 + t * K, K)])

        def orow(i, c):
            ones[i, :] = one16
            return c
        lax.fori_loop(0, K, orow, 0)

    plsc.subcore_barrier()

    pltpu.sync_copy(src_hbm.at[wid, 0], sidx.at[0])
    pltpu.sync_copy(dst_hbm.at[wid, 0], didx.at[0])
    pltpu.async_copy(y_hbm.at[sidx.at[0]], rows.at[0], sem)

    def chunk(j, c):
        par = lax.rem(j, 2)
        nxt = lax.rem(j + 1, 2)

        @pl.when(j + 1 < NCHUNK)
        def _():
            pltpu.sync_copy(src_hbm.at[wid, j + 1], sidx.at[nxt])
            pltpu.sync_copy(dst_hbm.at[wid, j + 1], didx.at[nxt])

        pltpu.make_async_copy(y_hbm.at[sidx.at[par]], rows.at[par], sem).wait()

        @pl.when(j + 1 < NCHUNK)
        def _():
            pltpu.async_copy(y_hbm.at[sidx.at[nxt]], rows.at[nxt], sem)

        if with_counts:
            cdesc = pltpu.async_copy(ones, cacc.at[didx.at[par]], csem, add=True)
        pltpu.sync_copy(rows.at[par], acc.at[didx.at[par]], add=True)
        if with_counts:
            cdesc.wait()
        return c
    lax.fori_loop(0, NCHUNK, chunk, 0)

    plsc.subcore_barrier()

    pltpu.sync_copy(acc.at[pl.ds(r0, RPT)], out_hbm.at[core, pl.ds(r0, RPT)])
    if with_counts:
        pltpu.sync_copy(cacc.at[pl.ds(r0, RPT)], cnt_hbm.at[core, pl.ds(r0, RPT)])


def _make_agg(with_counts):
    mesh = plsc.VectorSubcoreMesh(core_axis_name="c", subcore_axis_name="s")
    out_type = [jax.ShapeDtypeStruct((NC, NP, D), jnp.float32)]
    scratch = [
        pltpu.VMEM((2, K), jnp.int32),
        pltpu.VMEM((2, K), jnp.int32),
        pltpu.VMEM((2, K, D), jnp.float32),
        pltpu.VMEM_SHARED((NP, D), jnp.float32),
        pltpu.SemaphoreType.DMA,
    ]
    if with_counts:
        out_type.append(jax.ShapeDtypeStruct((NC, NP, CW), jnp.float32))
        scratch += [
            pltpu.SemaphoreType.DMA,
            pltpu.VMEM((K, CW), jnp.float32),
            pltpu.VMEM((K, CW), jnp.float32),
            pltpu.VMEM_SHARED((NP, CW), jnp.float32),
        ]
    return pl.kernel(
        functools.partial(_agg_body, with_counts),
        out_type=out_type,
        mesh=mesh,
        scratch_types=scratch,
        compiler_params=pltpu.CompilerParams(use_tc_tiling_on_sc=False),
    )


_agg_with_counts = _make_agg(True)
_agg_no_counts = _make_agg(False)


_RB = 1000
_NG = N // _RB

_full_w = pl.BlockSpec((D, D), lambda i: (0, 0))
_full_b = pl.BlockSpec((1, D), lambda i: (0, 0))
_row_blk = pl.BlockSpec((_RB, D), lambda i: (i, 0))
_agg_blk = pl.BlockSpec((NC, _RB, D), lambda i: (0, i, 0))
_cnt_blk = pl.BlockSpec((NC, _RB, CW), lambda i: (0, i, 0))


def _pre_body(x_ref, wl_ref, wr_ref, b_ref, y_ref, s_ref):
    xb = x_ref[...]
    y_ref[...] = jnp.dot(xb, wl_ref[...], preferred_element_type=jnp.float32)
    s_ref[...] = (jnp.dot(xb, wr_ref[...], preferred_element_type=jnp.float32)
                  + b_ref[...])


def _pre(x, wl, wr, b):
    return pl.pallas_call(
        _pre_body,
        grid=(_NG,),
        in_specs=[_row_blk, _full_w, _full_w, _full_b],
        out_specs=[_row_blk, _row_blk],
        out_shape=[jax.ShapeDtypeStruct((N, D), jnp.float32),
                   jax.ShapeDtypeStruct((N, D), jnp.float32)],
    )(x, wl, wr, b)


def _mid_body(agg_ref, cnt_ref, s_ref, wl_ref, wr_ref, b_ref, y_ref, s2_ref):
    a = agg_ref[0] + agg_ref[1]
    cn = cnt_ref[0, :, 0:1] + cnt_ref[1, :, 0:1]
    rinv = 1.0 / jnp.maximum(cn, 1.0)
    z = jnp.maximum(a * rinv + s_ref[...], 0.0)
    y_ref[...] = jnp.dot(z, wl_ref[...], preferred_element_type=jnp.float32)
    s2_ref[...] = (jnp.dot(z, wr_ref[...], preferred_element_type=jnp.float32)
                   + b_ref[...])


def _mid(agg, cnt, s1, wl, wr, b):
    return pl.pallas_call(
        _mid_body,
        grid=(_NG,),
        in_specs=[_agg_blk, _cnt_blk, _row_blk, _full_w, _full_w, _full_b],
        out_specs=[_row_blk, _row_blk],
        out_shape=[jax.ShapeDtypeStruct((N, D), jnp.float32),
                   jax.ShapeDtypeStruct((N, D), jnp.float32)],
    )(agg, cnt, s1, wl, wr, b)


def _fin_body(agg_ref, cnt_ref, s_ref, o_ref):
    a = agg_ref[0] + agg_ref[1]
    cn = cnt_ref[0, :, 0:1] + cnt_ref[1, :, 0:1]
    rinv = 1.0 / jnp.maximum(cn, 1.0)
    o_ref[...] = a * rinv + s_ref[...]


def _fin(agg, cnt, s2):
    return pl.pallas_call(
        _fin_body,
        grid=(_NG,),
        in_specs=[_agg_blk, _cnt_blk, _row_blk],
        out_specs=_row_blk,
        out_shape=jax.ShapeDtypeStruct((N, D), jnp.float32),
    )(agg, cnt, s2)


@jax.jit
def kernel(x, edge_index, W_l1, b_l1, W_r1, W_l2, b_l2, W_r2):
    src = edge_index[0].reshape(NW, NCHUNK, K)
    dst = edge_index[1].reshape(NW, NCHUNK, K)
    y1, s1 = _pre(x, W_l1, W_r1, b_l1.reshape(1, D))
    agg1, cnt = _agg_with_counts(y1, src, dst)
    y2, s2 = _mid(agg1, cnt, s1, W_l2, W_r2, b_l2.reshape(1, D))
    (agg2,) = _agg_no_counts(y2, src, dst)
    return _fin(agg2, cnt, s2)
